# baseline (speedup 1.0000x reference)
import functools
import math

import jax
import jax.numpy as jnp
from jax import lax
from jax.experimental import pallas as pl
from jax.experimental.pallas import tpu as pltpu
from jax.experimental.pallas import tpu_sc as plsc

B, S, D, H, E = 2, 2048, 768, 1536, 64
T = B * S
BT = 256
NT = 80
XS_PAD = NT * BT
PW = 128

NC, NS = 2, 16
NW = NC * NS
PER_W = T // NW
CH = 64


def _router_body(x_ref, wg_ref, bg_ref, pos_ref, pw_ref, eid_ref):
    x = x_ref[...]
    logits = jnp.dot(x, wg_ref[...], preferred_element_type=jnp.float32)
    logits = logits + bg_ref[...]
    m = jnp.max(logits, axis=1, keepdims=True)
    iota_e = lax.broadcasted_iota(jnp.int32, (T, E), 1)
    top1 = jnp.min(jnp.where(logits == m, iota_e, E), axis=1, keepdims=True)
    pmax = 1.0 / jnp.sum(jnp.exp(logits - m), axis=1, keepdims=True)

    onehot = (iota_e == top1).astype(jnp.int32)
    inc = onehot
    k = 1
    while k < T:
        shifted = jnp.concatenate(
            [jnp.zeros((k, E), jnp.int32), inc[: T - k, :]], axis=0)
        inc = inc + shifted
        k *= 2
    rank = jnp.sum(onehot * inc, axis=1, keepdims=True) - 1
    counts = inc[T - 1:T, :]

    ntiles = (counts + BT - 1) // BT
    cum = ntiles
    k = 1
    while k < E:
        shifted = jnp.concatenate(
            [jnp.zeros((1, k), jnp.int32), cum[:, : E - k]], axis=1)
        cum = cum + shifted
        k *= 2
    excl = cum - ntiles
    start_tok = jnp.sum(onehot * (BT * excl), axis=1, keepdims=True)
    pos_ref[...] = start_tok + rank

    pw_ref[...] = jnp.broadcast_to(pmax, (T, PW))

    ii = lax.broadcasted_iota(jnp.int32, (NT, E), 0)
    cum_b = jnp.broadcast_to(cum, (NT, E))
    eid = jnp.sum((cum_b <= ii).astype(jnp.int32), axis=1, keepdims=True)
    eid_ref[...] = jnp.minimum(eid, E - 1)


def _run_router(xf, Wg, bg):
    return pl.pallas_call(
        _router_body,
        out_shape=[
            jax.ShapeDtypeStruct((T, 1), jnp.int32),
            jax.ShapeDtypeStruct((T, PW), jnp.float32),
            jax.ShapeDtypeStruct((NT, 1), jnp.int32),
        ],
    )(xf, Wg, bg.reshape(1, E))


def _dispatch_body(pos_hbm, xf_hbm, pw_hbm, xs_hbm, ps_hbm,
                   idx_v, rows_v, prow_v, sem_x, sem_p):
    wid = lax.axis_index("s") * NC + lax.axis_index("c")
    for c in range(PER_W // CH):
        base = wid * PER_W + c * CH
        pltpu.sync_copy(pos_hbm.at[pl.ds(base, CH)], idx_v)
        pltpu.sync_copy(xf_hbm.at[pl.ds(base, CH)], rows_v)
        cx = pltpu.async_copy(rows_v, xs_hbm.at[idx_v], sem_x)
        pltpu.sync_copy(pw_hbm.at[pl.ds(base, CH)], prow_v)
        cp = pltpu.async_copy(prow_v, ps_hbm.at[idx_v], sem_p)
        cx.wait()
        cp.wait()


def _run_dispatch(pos, xf, pw):
    f = functools.partial(
        pl.kernel,
        out_type=[
            jax.ShapeDtypeStruct((XS_PAD, D), jnp.float32),
            jax.ShapeDtypeStruct((XS_PAD, PW), jnp.float32),
        ],
        mesh=plsc.VectorSubcoreMesh(core_axis_name="c", subcore_axis_name="s"),
        scratch_types=[
            pltpu.VMEM((CH,), jnp.int32),
            pltpu.VMEM((CH, D), jnp.float32),
            pltpu.VMEM((CH, PW), jnp.float32),
            pltpu.SemaphoreType.DMA,
            pltpu.SemaphoreType.DMA,
        ],
    )(_dispatch_body)
    return f(pos, xf, pw)


def _ffn_body(eids, xs_ref, ps_ref, w1_ref, b1_ref, w2_ref, b2_ref, os_ref):
    x = xs_ref[...].astype(jnp.bfloat16)
    h = jnp.dot(x, w1_ref[0], preferred_element_type=jnp.float32)
    h = h + b1_ref[0]
    h = 0.5 * h * (1.0 + lax.erf(h * (1.0 / math.sqrt(2.0))))
    o = jnp.dot(h.astype(jnp.bfloat16), w2_ref[0],
                preferred_element_type=jnp.float32)
    o = o + b2_ref[0]
    os_ref[...] = o * ps_ref[:, 0:1]


def _run_ffn(eids, xs, ps, W1, b1, W2, b2):
    grid_spec = pltpu.PrefetchScalarGridSpec(
        num_scalar_prefetch=1,
        grid=(NT,),
        in_specs=[
            pl.BlockSpec((BT, D), lambda i, eids: (i, 0)),
            pl.BlockSpec((BT, PW), lambda i, eids: (i, 0)),
            pl.BlockSpec((1, D, H), lambda i, eids: (eids[i], 0, 0)),
            pl.BlockSpec((1, 1, H), lambda i, eids: (eids[i], 0, 0)),
            pl.BlockSpec((1, H, D), lambda i, eids: (eids[i], 0, 0)),
            pl.BlockSpec((1, 1, D), lambda i, eids: (eids[i], 0, 0)),
        ],
        out_specs=pl.BlockSpec((BT, D), lambda i, eids: (i, 0)),
    )
    return pl.pallas_call(
        _ffn_body,
        grid_spec=grid_spec,
        out_shape=jax.ShapeDtypeStruct((XS_PAD, D), jnp.float32),
    )(eids, xs, ps, W1.astype(jnp.bfloat16), b1.reshape(E, 1, H),
      W2.astype(jnp.bfloat16), b2.reshape(E, 1, D))


def _combine_body(pos_hbm, os_hbm, out_hbm, idx_v, rows_v, sem):
    wid = lax.axis_index("s") * NC + lax.axis_index("c")
    for c in range(PER_W // CH):
        base = wid * PER_W + c * CH
        pltpu.sync_copy(pos_hbm.at[pl.ds(base, CH)], idx_v)
        pltpu.async_copy(os_hbm.at[idx_v], rows_v, sem).wait()
        pltpu.sync_copy(rows_v, out_hbm.at[pl.ds(base, CH)])


def _run_combine(pos, os):
    f = functools.partial(
        pl.kernel,
        out_type=jax.ShapeDtypeStruct((T, D), jnp.float32),
        mesh=plsc.VectorSubcoreMesh(core_axis_name="c", subcore_axis_name="s"),
        scratch_types=[
            pltpu.VMEM((CH,), jnp.int32),
            pltpu.VMEM((CH, D), jnp.float32),
            pltpu.SemaphoreType.DMA,
        ],
    )(_combine_body)
    return f(pos, os)


@jax.jit
def kernel(x, W1, b1, W2, b2, Wg, bg):
    xf = x.reshape(T, D)
    pos2, pw, eids2 = _run_router(xf, Wg, bg)
    pos = pos2.reshape(T)
    eids = eids2.reshape(NT)
    xs, ps = _run_dispatch(pos, xf, pw)
    os = _run_ffn(eids, xs, ps, W1, b1, W2, b2)
    out = _run_combine(pos, os)
    return out.reshape(B, S, D)


# speedup vs baseline: 1.7713x; 1.7713x over previous
import functools
import math

import jax
import jax.numpy as jnp
from jax import lax
from jax.experimental import pallas as pl
from jax.experimental.pallas import tpu as pltpu
from jax.experimental.pallas import tpu_sc as plsc

B, S, D, H, E = 2, 2048, 768, 1536, 64
T = B * S
BT = 128
NT = 96
XS_PAD = NT * BT
PW = 128

NC, NS = 2, 16
NW = NC * NS
PER_W = T // NW
CH = 64


def _router_body(x_ref, wg_ref, bg_ref, pos_ref, pw_ref, eid_ref, blk_ref):
    x = x_ref[...]
    logits = jnp.dot(x, wg_ref[...], preferred_element_type=jnp.float32)
    logits = logits + bg_ref[...]
    m = jnp.max(logits, axis=1, keepdims=True)
    iota_e = lax.broadcasted_iota(jnp.int32, (T, E), 1)
    top1 = jnp.min(jnp.where(logits == m, iota_e, E), axis=1, keepdims=True)
    pmax = 1.0 / jnp.sum(jnp.exp(logits - m), axis=1, keepdims=True)

    onehot = (iota_e == top1).astype(jnp.int32)
    inc = onehot
    k = 1
    while k < T:
        shifted = jnp.concatenate(
            [jnp.zeros((k, E), jnp.int32), inc[: T - k, :]], axis=0)
        inc = inc + shifted
        k *= 2
    rank = jnp.sum(onehot * inc, axis=1, keepdims=True) - 1
    counts = inc[T - 1:T, :]

    ntiles = (counts + BT - 1) // BT
    cum = ntiles
    k = 1
    while k < E:
        shifted = jnp.concatenate(
            [jnp.zeros((1, k), jnp.int32), cum[:, : E - k]], axis=1)
        cum = cum + shifted
        k *= 2
    excl = cum - ntiles
    start_tok = jnp.sum(onehot * (BT * excl), axis=1, keepdims=True)
    pos_ref[...] = start_tok + rank

    pw_ref[...] = jnp.broadcast_to(pmax, (T, PW))

    ii = lax.broadcasted_iota(jnp.int32, (NT, E), 0)
    cum_b = jnp.broadcast_to(cum, (NT, E))
    eid = jnp.sum((cum_b <= ii).astype(jnp.int32), axis=1, keepdims=True)
    iota_e_row = lax.broadcasted_iota(jnp.int32, (1, E), 1)
    last_e = jnp.max(jnp.where(counts > 0, iota_e_row, 0))
    eid_ref[...] = jnp.minimum(eid, last_e)
    total = cum[0, E - 1]
    ii1 = lax.broadcasted_iota(jnp.int32, (NT, 1), 0)
    blk_ref[...] = jnp.where(ii1 < total, ii1, total - 1)


def _run_router(xf, Wg, bg):
    return pl.pallas_call(
        _router_body,
        out_shape=[
            jax.ShapeDtypeStruct((T, 1), jnp.int32),
            jax.ShapeDtypeStruct((T, PW), jnp.float32),
            jax.ShapeDtypeStruct((NT, 1), jnp.int32),
            jax.ShapeDtypeStruct((NT, 1), jnp.int32),
        ],
    )(xf, Wg, bg.reshape(1, E))


def _dispatch_body(pos_hbm, xf_hbm, pw_hbm, xs_hbm, ps_hbm,
                   idx_v, rows_v, prow_v, sem_x, sem_p):
    wid = lax.axis_index("s") * NC + lax.axis_index("c")
    for c in range(PER_W // CH):
        base = wid * PER_W + c * CH
        pltpu.sync_copy(pos_hbm.at[pl.ds(base, CH)], idx_v)
        pltpu.sync_copy(xf_hbm.at[pl.ds(base, CH)], rows_v)
        cx = pltpu.async_copy(rows_v, xs_hbm.at[idx_v], sem_x)
        pltpu.sync_copy(pw_hbm.at[pl.ds(base, CH)], prow_v)
        cp = pltpu.async_copy(prow_v, ps_hbm.at[idx_v], sem_p)
        cx.wait()
        cp.wait()


def _run_dispatch(pos, xf, pw):
    f = functools.partial(
        pl.kernel,
        out_type=[
            jax.ShapeDtypeStruct((XS_PAD, D), jnp.float32),
            jax.ShapeDtypeStruct((XS_PAD, PW), jnp.float32),
        ],
        mesh=plsc.VectorSubcoreMesh(core_axis_name="c", subcore_axis_name="s"),
        scratch_types=[
            pltpu.VMEM((CH,), jnp.int32),
            pltpu.VMEM((CH, D), jnp.float32),
            pltpu.VMEM((CH, PW), jnp.float32),
            pltpu.SemaphoreType.DMA,
            pltpu.SemaphoreType.DMA,
        ],
    )(_dispatch_body)
    return f(pos, xf, pw)


def _ffn_body(eids, blks, xs_ref, ps_ref, w1_ref, b1_ref, w2_ref, b2_ref,
              os_ref):
    x = xs_ref[...]
    h = jnp.dot(x, w1_ref[0], preferred_element_type=jnp.float32)
    h = h + b1_ref[0]
    h = 0.5 * h * (1.0 + lax.erf(h * (1.0 / math.sqrt(2.0))))
    o = jnp.dot(h, w2_ref[0], preferred_element_type=jnp.float32)
    o = o + b2_ref[0]
    os_ref[...] = o * ps_ref[:, 0:1]


def _run_ffn(eids, blks, xs, ps, W1, b1, W2, b2):
    grid_spec = pltpu.PrefetchScalarGridSpec(
        num_scalar_prefetch=2,
        grid=(NT,),
        in_specs=[
            pl.BlockSpec((BT, D), lambda i, eids, blks: (blks[i], 0)),
            pl.BlockSpec((BT, PW), lambda i, eids, blks: (blks[i], 0)),
            pl.BlockSpec((1, D, H), lambda i, eids, blks: (eids[i], 0, 0)),
            pl.BlockSpec((1, 1, H), lambda i, eids, blks: (eids[i], 0, 0)),
            pl.BlockSpec((1, H, D), lambda i, eids, blks: (eids[i], 0, 0)),
            pl.BlockSpec((1, 1, D), lambda i, eids, blks: (eids[i], 0, 0)),
        ],
        out_specs=pl.BlockSpec((BT, D), lambda i, eids, blks: (blks[i], 0)),
    )
    return pl.pallas_call(
        _ffn_body,
        grid_spec=grid_spec,
        out_shape=jax.ShapeDtypeStruct((XS_PAD, D), jnp.float32),
    )(eids, blks, xs, ps, W1, b1.reshape(E, 1, H), W2, b2.reshape(E, 1, D))


def _combine_body(pos_hbm, os_hbm, out_hbm, idx_v, rows_v, sem):
    wid = lax.axis_index("s") * NC + lax.axis_index("c")
    for c in range(PER_W // CH):
        base = wid * PER_W + c * CH
        pltpu.sync_copy(pos_hbm.at[pl.ds(base, CH)], idx_v)
        pltpu.async_copy(os_hbm.at[idx_v], rows_v, sem).wait()
        pltpu.sync_copy(rows_v, out_hbm.at[pl.ds(base, CH)])


def _run_combine(pos, os):
    f = functools.partial(
        pl.kernel,
        out_type=jax.ShapeDtypeStruct((T, D), jnp.float32),
        mesh=plsc.VectorSubcoreMesh(core_axis_name="c", subcore_axis_name="s"),
        scratch_types=[
            pltpu.VMEM((CH,), jnp.int32),
            pltpu.VMEM((CH, D), jnp.float32),
            pltpu.SemaphoreType.DMA,
        ],
    )(_combine_body)
    return f(pos, os)


@jax.jit
def kernel(x, W1, b1, W2, b2, Wg, bg):
    xf = x.reshape(T, D)
    pos2, pw, eids2, blks2 = _run_router(xf, Wg, bg)
    pos = pos2.reshape(T)
    eids = eids2.reshape(NT)
    blks = blks2.reshape(NT)
    xs, ps = _run_dispatch(pos, xf, pw)
    os = _run_ffn(eids, blks, xs, ps, W1, b1, W2, b2)
    out = _run_combine(pos, os)
    return out.reshape(B, S, D)


# speedup vs baseline: 1.7790x; 1.0044x over previous
import functools
import math

import jax
import jax.numpy as jnp
from jax import lax
from jax.experimental import pallas as pl
from jax.experimental.pallas import tpu as pltpu
from jax.experimental.pallas import tpu_sc as plsc

B, S, D, H, E = 2, 2048, 768, 1536, 64
T = B * S
BT = 128
NT = 96
XS_PAD = NT * BT
PW = 128

NC, NS = 2, 16
NW = NC * NS
PER_W = T // NW
CH = 64


def _router_body(x_ref, wg_ref, bg_ref, pos_ref, pw_ref, eid_ref, blk_ref):
    x = x_ref[...]
    logits = jnp.dot(x, wg_ref[...], preferred_element_type=jnp.float32)
    logits = logits + bg_ref[...]
    m = jnp.max(logits, axis=1, keepdims=True)
    iota_e = lax.broadcasted_iota(jnp.int32, (T, E), 1)
    top1 = jnp.min(jnp.where(logits == m, iota_e, E), axis=1, keepdims=True)
    pmax = 1.0 / jnp.sum(jnp.exp(logits - m), axis=1, keepdims=True)

    onehot = (iota_e == top1).astype(jnp.int32)
    inc = onehot
    k = 1
    while k < T:
        shifted = jnp.concatenate(
            [jnp.zeros((k, E), jnp.int32), inc[: T - k, :]], axis=0)
        inc = inc + shifted
        k *= 2
    rank = jnp.sum(onehot * inc, axis=1, keepdims=True) - 1
    counts = inc[T - 1:T, :]

    ntiles = (counts + BT - 1) // BT
    cum = ntiles
    k = 1
    while k < E:
        shifted = jnp.concatenate(
            [jnp.zeros((1, k), jnp.int32), cum[:, : E - k]], axis=1)
        cum = cum + shifted
        k *= 2
    excl = cum - ntiles
    start_tok = jnp.sum(onehot * (BT * excl), axis=1, keepdims=True)
    pos_ref[...] = (start_tok + rank).reshape(T)

    pw_ref[...] = jnp.broadcast_to(pmax, (T, PW))

    ii = lax.broadcasted_iota(jnp.int32, (NT, E), 0)
    cum_b = jnp.broadcast_to(cum, (NT, E))
    eid = jnp.sum((cum_b <= ii).astype(jnp.int32), axis=1, keepdims=True)
    iota_e_row = lax.broadcasted_iota(jnp.int32, (1, E), 1)
    last_e = jnp.max(jnp.where(counts > 0, iota_e_row, 0))
    eid_ref[...] = jnp.minimum(eid, last_e).reshape(NT)
    total = cum[0, E - 1]
    ii1 = lax.broadcasted_iota(jnp.int32, (NT, 1), 0)
    blk_ref[...] = jnp.where(ii1 < total, ii1, total - 1).reshape(NT)


def _run_router(xf, Wg, bg):
    return pl.pallas_call(
        _router_body,
        out_shape=[
            jax.ShapeDtypeStruct((T,), jnp.int32),
            jax.ShapeDtypeStruct((T, PW), jnp.float32),
            jax.ShapeDtypeStruct((NT,), jnp.int32),
            jax.ShapeDtypeStruct((NT,), jnp.int32),
        ],
    )(xf, Wg, bg.reshape(1, E))


def _dispatch_body(pos_hbm, xf_hbm, pw_hbm, xs_hbm, ps_hbm,
                   idx_v, rows_v, prow_v, sem_x, sem_p):
    wid = lax.axis_index("s") * NC + lax.axis_index("c")
    for c in range(PER_W // CH):
        base = wid * PER_W + c * CH
        pltpu.sync_copy(pos_hbm.at[pl.ds(base, CH)], idx_v)
        pltpu.sync_copy(xf_hbm.at[pl.ds(base, CH)], rows_v)
        cx = pltpu.async_copy(rows_v, xs_hbm.at[idx_v], sem_x)
        pltpu.sync_copy(pw_hbm.at[pl.ds(base, CH)], prow_v)
        cp = pltpu.async_copy(prow_v, ps_hbm.at[idx_v], sem_p)
        cx.wait()
        cp.wait()


def _run_dispatch(pos, xf, pw):
    f = functools.partial(
        pl.kernel,
        out_type=[
            jax.ShapeDtypeStruct((XS_PAD, D), jnp.float32),
            jax.ShapeDtypeStruct((XS_PAD, PW), jnp.float32),
        ],
        mesh=plsc.VectorSubcoreMesh(core_axis_name="c", subcore_axis_name="s"),
        scratch_types=[
            pltpu.VMEM((CH,), jnp.int32),
            pltpu.VMEM((CH, D), jnp.float32),
            pltpu.VMEM((CH, PW), jnp.float32),
            pltpu.SemaphoreType.DMA,
            pltpu.SemaphoreType.DMA,
        ],
    )(_dispatch_body)
    return f(pos, xf, pw)


def _ffn_body(eids, blks, xs_ref, ps_ref, w1_ref, b1_ref, w2_ref, b2_ref,
              os_ref):
    x = xs_ref[...]
    h = jnp.dot(x, w1_ref[0], preferred_element_type=jnp.float32)
    h = h + b1_ref[0]
    h = 0.5 * h * (1.0 + lax.erf(h * (1.0 / math.sqrt(2.0))))
    o = jnp.dot(h, w2_ref[0], preferred_element_type=jnp.float32)
    o = o + b2_ref[0]
    os_ref[...] = o * ps_ref[:, 0:1]


def _run_ffn(eids, blks, xs, ps, W1, b1, W2, b2):
    grid_spec = pltpu.PrefetchScalarGridSpec(
        num_scalar_prefetch=2,
        grid=(NT,),
        in_specs=[
            pl.BlockSpec((BT, D), lambda i, eids, blks: (blks[i], 0)),
            pl.BlockSpec((BT, PW), lambda i, eids, blks: (blks[i], 0)),
            pl.BlockSpec((1, D, H), lambda i, eids, blks: (eids[i], 0, 0)),
            pl.BlockSpec((1, 1, H), lambda i, eids, blks: (eids[i], 0, 0)),
            pl.BlockSpec((1, H, D), lambda i, eids, blks: (eids[i], 0, 0)),
            pl.BlockSpec((1, 1, D), lambda i, eids, blks: (eids[i], 0, 0)),
        ],
        out_specs=pl.BlockSpec((BT, D), lambda i, eids, blks: (blks[i], 0)),
    )
    return pl.pallas_call(
        _ffn_body,
        grid_spec=grid_spec,
        out_shape=jax.ShapeDtypeStruct((XS_PAD, D), jnp.float32),
    )(eids, blks, xs, ps, W1, b1.reshape(E, 1, H), W2, b2.reshape(E, 1, D))


def _combine_body(pos_hbm, os_hbm, out_hbm, idx_v, rows_v, sem):
    wid = lax.axis_index("s") * NC + lax.axis_index("c")
    for c in range(PER_W // CH):
        base = wid * PER_W + c * CH
        pltpu.sync_copy(pos_hbm.at[pl.ds(base, CH)], idx_v)
        pltpu.async_copy(os_hbm.at[idx_v], rows_v, sem).wait()
        pltpu.sync_copy(rows_v, out_hbm.at[pl.ds(base, CH)])


def _run_combine(pos, os):
    f = functools.partial(
        pl.kernel,
        out_type=jax.ShapeDtypeStruct((T, D), jnp.float32),
        mesh=plsc.VectorSubcoreMesh(core_axis_name="c", subcore_axis_name="s"),
        scratch_types=[
            pltpu.VMEM((CH,), jnp.int32),
            pltpu.VMEM((CH, D), jnp.float32),
            pltpu.SemaphoreType.DMA,
        ],
    )(_combine_body)
    return f(pos, os)


@jax.jit
def kernel(x, W1, b1, W2, b2, Wg, bg):
    xf = x.reshape(T, D)
    pos, pw, eids, blks = _run_router(xf, Wg, bg)
    xs, ps = _run_dispatch(pos, xf, pw)
    os = _run_ffn(eids, blks, xs, ps, W1, b1, W2, b2)
    out = _run_combine(pos, os)
    return out.reshape(B, S, D)


# speedup vs baseline: 1.8379x; 1.0331x over previous
import functools
import math

import jax
import jax.numpy as jnp
from jax import lax
from jax.experimental import pallas as pl
from jax.experimental.pallas import tpu as pltpu
from jax.experimental.pallas import tpu_sc as plsc

B, S, D, H, E = 2, 2048, 768, 1536, 64
T = B * S
BT = 128
NT = 96
XS_PAD = NT * BT
XH = D // 2
XW = XH + 128

NC, NS = 2, 16
NW = NC * NS
PER_W = T // NW
CH = 64


def _router_body(x_ref, wg_ref, bg_ref, pos_ref, pw_ref, eid_ref, blk_ref):
    x = x_ref[...]
    logits = jnp.dot(x, wg_ref[...], preferred_element_type=jnp.float32)
    logits = logits + bg_ref[...]
    m = jnp.max(logits, axis=1, keepdims=True)
    iota_e = lax.broadcasted_iota(jnp.int32, (T, E), 1)
    top1 = jnp.min(jnp.where(logits == m, iota_e, E), axis=1, keepdims=True)
    pmax = 1.0 / jnp.sum(jnp.exp(logits - m), axis=1, keepdims=True)

    onehot = (iota_e == top1).astype(jnp.int32)
    inc = onehot
    k = 1
    while k < T:
        shifted = jnp.concatenate(
            [jnp.zeros((k, E), jnp.int32), inc[: T - k, :]], axis=0)
        inc = inc + shifted
        k *= 2
    rank = jnp.sum(onehot * inc, axis=1, keepdims=True) - 1
    counts = inc[T - 1:T, :]

    ntiles = (counts + BT - 1) // BT
    cum = ntiles
    k = 1
    while k < E:
        shifted = jnp.concatenate(
            [jnp.zeros((1, k), jnp.int32), cum[:, : E - k]], axis=1)
        cum = cum + shifted
        k *= 2
    excl = cum - ntiles
    start_tok = jnp.sum(onehot * (BT * excl), axis=1, keepdims=True)
    pos_ref[...] = (start_tok + rank).reshape(T)

    x_bf = x.astype(jnp.bfloat16)
    lo = lax.bitcast_convert_type(x_bf[:, :XH], jnp.uint16).astype(jnp.uint32)
    hi = lax.bitcast_convert_type(x_bf[:, XH:], jnp.uint16).astype(jnp.uint32)
    xi = lo | (hi << 16)
    pb = lax.bitcast_convert_type(pmax.astype(jnp.bfloat16),
                                  jnp.uint16).astype(jnp.uint32)
    pi = jnp.broadcast_to(pb | (pb << 16), (T, XW - XH))
    pw_ref[...] = lax.bitcast_convert_type(
        jnp.concatenate([xi, pi], axis=1), jnp.int32)

    ii = lax.broadcasted_iota(jnp.int32, (NT, E), 0)
    cum_b = jnp.broadcast_to(cum, (NT, E))
    eid = jnp.sum((cum_b <= ii).astype(jnp.int32), axis=1, keepdims=True)
    iota_e_row = lax.broadcasted_iota(jnp.int32, (1, E), 1)
    last_e = jnp.max(jnp.where(counts > 0, iota_e_row, 0))
    eid_ref[...] = jnp.minimum(eid, last_e).reshape(NT)
    total = cum[0, E - 1]
    ii1 = lax.broadcasted_iota(jnp.int32, (NT, 1), 0)
    blk_ref[...] = jnp.where(ii1 < total, ii1, total - 1).reshape(NT)


def _run_router(xf, Wg, bg):
    return pl.pallas_call(
        _router_body,
        out_shape=[
            jax.ShapeDtypeStruct((T,), jnp.int32),
            jax.ShapeDtypeStruct((T, XW), jnp.int32),
            jax.ShapeDtypeStruct((NT,), jnp.int32),
            jax.ShapeDtypeStruct((NT,), jnp.int32),
        ],
    )(xf, Wg, bg.reshape(1, E))


def _dispatch_body(pos_hbm, pw_hbm, xs_hbm, idx_v, rows_v, sem_x):
    wid = lax.axis_index("s") * NC + lax.axis_index("c")
    base = wid * PER_W
    pltpu.sync_copy(pos_hbm.at[pl.ds(base, PER_W)], idx_v)
    pltpu.sync_copy(pw_hbm.at[pl.ds(base, PER_W)], rows_v)
    pltpu.async_copy(rows_v, xs_hbm.at[idx_v], sem_x).wait()


def _run_dispatch(pos, pw):
    f = functools.partial(
        pl.kernel,
        out_type=jax.ShapeDtypeStruct((XS_PAD, XW), jnp.int32),
        mesh=plsc.VectorSubcoreMesh(core_axis_name="c", subcore_axis_name="s"),
        scratch_types=[
            pltpu.VMEM((PER_W,), jnp.int32),
            pltpu.VMEM((PER_W, XW), jnp.int32),
            pltpu.SemaphoreType.DMA,
        ],
    )(_dispatch_body)
    return f(pos, pw)


def _ffn_body(eids, blks, xs_ref, w1_ref, b1_ref, w2_ref, b2_ref, os_ref):
    u = lax.bitcast_convert_type(xs_ref[...], jnp.uint32)
    xi = u[:, :XH]
    xlo = lax.bitcast_convert_type((xi & 0xFFFF).astype(jnp.uint16),
                                   jnp.bfloat16).astype(jnp.float32)
    xhi = lax.bitcast_convert_type((xi >> 16).astype(jnp.uint16),
                                   jnp.bfloat16).astype(jnp.float32)
    x = jnp.concatenate([xlo, xhi], axis=1)
    p = lax.bitcast_convert_type(
        (u[:, XH:XH + 1] & 0xFFFF).astype(jnp.uint16),
        jnp.bfloat16).astype(jnp.float32)
    h = jnp.dot(x, w1_ref[0], preferred_element_type=jnp.float32)
    h = h + b1_ref[0]
    h = 0.5 * h * (1.0 + lax.erf(h * (1.0 / math.sqrt(2.0))))
    o = jnp.dot(h, w2_ref[0], preferred_element_type=jnp.float32)
    o = o + b2_ref[0]
    os_ref[...] = o * p


def _run_ffn(eids, blks, xs, W1, b1, W2, b2):
    grid_spec = pltpu.PrefetchScalarGridSpec(
        num_scalar_prefetch=2,
        grid=(NT,),
        in_specs=[
            pl.BlockSpec((BT, XW), lambda i, eids, blks: (blks[i], 0)),
            pl.BlockSpec((1, D, H), lambda i, eids, blks: (eids[i], 0, 0)),
            pl.BlockSpec((1, 1, H), lambda i, eids, blks: (eids[i], 0, 0)),
            pl.BlockSpec((1, H, D), lambda i, eids, blks: (eids[i], 0, 0)),
            pl.BlockSpec((1, 1, D), lambda i, eids, blks: (eids[i], 0, 0)),
        ],
        out_specs=pl.BlockSpec((BT, D), lambda i, eids, blks: (blks[i], 0)),
    )
    return pl.pallas_call(
        _ffn_body,
        grid_spec=grid_spec,
        out_shape=jax.ShapeDtypeStruct((XS_PAD, D), jnp.float32),
    )(eids, blks, xs, W1, b1.reshape(E, 1, H), W2, b2.reshape(E, 1, D))


def _combine_body(pos_hbm, os_hbm, out_hbm, idx_v, rows_v, sem):
    wid = lax.axis_index("s") * NC + lax.axis_index("c")
    for c in range(PER_W // CH):
        base = wid * PER_W + c * CH
        pltpu.sync_copy(pos_hbm.at[pl.ds(base, CH)], idx_v)
        pltpu.async_copy(os_hbm.at[idx_v], rows_v, sem).wait()
        pltpu.sync_copy(rows_v, out_hbm.at[pl.ds(base, CH)])


def _run_combine(pos, os):
    f = functools.partial(
        pl.kernel,
        out_type=jax.ShapeDtypeStruct((T, D), jnp.float32),
        mesh=plsc.VectorSubcoreMesh(core_axis_name="c", subcore_axis_name="s"),
        scratch_types=[
            pltpu.VMEM((CH,), jnp.int32),
            pltpu.VMEM((CH, D), jnp.float32),
            pltpu.SemaphoreType.DMA,
        ],
    )(_combine_body)
    return f(pos, os)


@jax.jit
def kernel(x, W1, b1, W2, b2, Wg, bg):
    xf = x.reshape(T, D)
    pos, pw, eids, blks = _run_router(xf, Wg, bg)
    xs = _run_dispatch(pos, pw)
    os = _run_ffn(eids, blks, xs, W1, b1, W2, b2)
    out = _run_combine(pos, os)
    return out.reshape(B, S, D)
